# baseline (speedup 1.0000x reference)
import functools

import jax
import jax.numpy as jnp
from jax import lax
from jax.experimental import pallas as pl
from jax.experimental.pallas import tpu as pltpu
from jax.experimental.pallas import tpu_sc as plsc


_F32 = jnp.float32
_I32 = jnp.int32

_NUM_CORES = 2
_NUM_SUBCORES = 16
_NW = _NUM_CORES * _NUM_SUBCORES
_CH = 400


def _prologue_body(bn_ref, ts_ref, el_ref, rid_ref, pid_ref,
                   i0_ref, i1_ref, i2_ref, i3_ref, n0_ref, n1_ref):
    t = ts_ref[...]
    rows, cols = t.shape
    lanes = lax.broadcasted_iota(_I32, (rows, cols), 1)
    tp = jnp.where(lanes >= 1, jnp.roll(t, 1, axis=1), t)
    m = jnp.where(t != tp, tp, -1.0)
    k = 1
    while k < cols:
        m = jnp.maximum(m, jnp.where(lanes >= k, jnp.roll(m, k, axis=1), -1.0))
        k *= 2
    prev = jnp.where(m < 0.0, t, m)
    lag = jnp.clip((t - prev) / 60000.0, 0.0, 1440.0)
    lag_cat = jnp.where(
        lag < 6.0, lag.astype(_I32), ((lag - 1.0) / 10.0).astype(_I32) + 6
    )
    e = el_ref[...]
    el_cat = jnp.clip(e.astype(_I32) + 1, 0, 300)
    i0_ref[...] = rid_ref[...]
    i1_ref[...] = lag_cat + 4
    i2_ref[...] = el_cat + 155
    i3_ref[...] = pid_ref[...] + 457
    e_num = jnp.clip(e, 0.0, 300.0)
    lf = jnp.log1p(lag)
    s0 = jnp.sqrt(bn_ref[1, 0] + 1e-5)
    s1 = jnp.sqrt(bn_ref[1, 1] + 1e-5)
    n0_ref[...] = (lf - bn_ref[0, 0]) / s0 * bn_ref[2, 0] + bn_ref[3, 0]
    n1_ref[...] = (e_num - bn_ref[0, 1]) / s1 * bn_ref[2, 1] + bn_ref[3, 1]


def _sc_gather(idx_all, fused_t):
    nch, _, ch = idx_all.shape
    emb = fused_t.shape[1]
    ch_per_w = nch // _NW
    mesh = plsc.VectorSubcoreMesh(
        core_axis_name="c", subcore_axis_name="s",
        num_cores=_NUM_CORES, num_subcores=_NUM_SUBCORES,
    )
    out_type = jax.ShapeDtypeStruct((nch, 4, ch, emb), _F32)
    scratch = [
        pltpu.VMEM((4, ch), _I32),
        pltpu.VMEM((4, ch, emb), _F32),
        pltpu.SemaphoreType.DMA,
    ]

    @functools.partial(pl.kernel, mesh=mesh, out_type=out_type,
                       scratch_types=scratch,
                       compiler_params=pltpu.CompilerParams(
                           use_tc_tiling_on_sc=False))
    def body(idx_h, ft_h, out_h, idx_v, buf_v, sem):
        wid = lax.axis_index("s") * _NUM_CORES + lax.axis_index("c")
        base = wid * ch_per_w

        def step(j, carry):
            pltpu.sync_copy(idx_h.at[base + j], idx_v)
            cps = [pltpu.async_copy(ft_h.at[idx_v.at[t]], buf_v.at[t], sem)
                   for t in range(4)]
            for c in cps:
                c.wait()
            pltpu.sync_copy(buf_v, out_h.at[base + j])
            return carry

        lax.fori_loop(0, ch_per_w, step, 0)

    return body(idx_all, fused_t)


def _epilogue_body(wr_ref, wn_ref, wl_ref, we_ref, wp_ref, nw_ref, nb_ref,
                   lb_ref, g_ref, bb_ref, gath_ref, nm_ref, out_ref):
    blk = out_ref.shape[0]
    emb = wl_ref.shape[0]
    numemb = (nm_ref[:, 0:1] * nw_ref[0:1, :]
              + nm_ref[:, 1:2] * nw_ref[1:2, :] + nb_ref[...])
    resp = gath_ref[:, 0].reshape(blk, emb)
    lagg = gath_ref[:, 1].reshape(blk, emb)
    elg = gath_ref[:, 2].reshape(blk, emb)
    posg = gath_ref[:, 3].reshape(blk, emb)
    y = jnp.dot(resp, wr_ref[...], preferred_element_type=_F32)
    y = y + jnp.dot(numemb, wn_ref[...], preferred_element_type=_F32)
    y = y + jnp.dot(lagg, wl_ref[...], preferred_element_type=_F32)
    y = y + jnp.dot(elg, we_ref[...], preferred_element_type=_F32)
    y = y + jnp.dot(posg, wp_ref[...], preferred_element_type=_F32)
    y = y + lb_ref[...]
    mu = jnp.mean(y, axis=1, keepdims=True)
    d = y - mu
    var = jnp.mean(d * d, axis=1, keepdims=True)
    out_ref[...] = d / jnp.sqrt(var + 1e-12) * g_ref[...] + bb_ref[...]


def kernel(input_ids, position_ids, timestamp, elapsed_time, response_table,
           num_W, num_b, bn_gamma, bn_beta, bn_mean, bn_var, elapsed_table,
           lag_table, pos_table, lin_W, lin_b, ln_gamma, ln_beta):
    b, l = input_ids.shape
    n = b * l
    hid = lin_W.shape[1]
    resp_w = response_table.shape[1]
    emb = lag_table.shape[1]
    nch = n // _CH

    ts_f = timestamp.astype(_F32)
    bn = jnp.stack([bn_mean.astype(_F32), bn_var.astype(_F32),
                    bn_gamma.astype(_F32), bn_beta.astype(_F32)], axis=0)
    rb = 256
    bspec = pl.BlockSpec((rb, l), lambda i: (i, 0))
    i0, i1, i2, i3, n0, n1 = pl.pallas_call(
        _prologue_body,
        grid=(b // rb,),
        in_specs=[pl.BlockSpec((4, 2), lambda i: (0, 0))] + [bspec] * 4,
        out_specs=[bspec] * 6,
        out_shape=[jax.ShapeDtypeStruct((b, l), _I32)] * 4
        + [jax.ShapeDtypeStruct((b, l), _F32)] * 2,
    )(bn, ts_f, elapsed_time.astype(_F32), input_ids.astype(_I32),
      position_ids.astype(_I32))

    idx_all = jnp.stack(
        [x.reshape(nch, _CH) for x in (i0, i1, i2, i3)], axis=1)
    fused_t = jnp.concatenate([
        jnp.pad(response_table.astype(_F32), ((0, 0), (0, emb - resp_w))),
        lag_table.astype(_F32), elapsed_table.astype(_F32),
        pos_table.astype(_F32)], axis=0)
    gath = _sc_gather(idx_all, fused_t)

    nm = jnp.stack([n0.reshape(n), n1.reshape(n)], axis=-1)
    wr = jnp.pad(lin_W[0:resp_w], ((0, emb - resp_w), (0, 0)))
    wn = lin_W[resp_w:resp_w + emb]
    wl = lin_W[resp_w + emb:resp_w + 2 * emb]
    we = lin_W[resp_w + 2 * emb:resp_w + 3 * emb]
    wp = lin_W[resp_w + 3 * emb:resp_w + 4 * emb]

    kb = 2
    blk = kb * _CH
    const = lambda shape: pl.BlockSpec(shape, lambda i: (0, 0))
    weights = (wr, wn, wl, we, wp, num_W.astype(_F32), num_b.reshape(1, emb),
               lin_b.reshape(1, hid), ln_gamma.reshape(1, hid),
               ln_beta.reshape(1, hid))
    out = pl.pallas_call(
        _epilogue_body,
        grid=(n // blk,),
        in_specs=[const(w.shape) for w in weights]
        + [pl.BlockSpec((kb, 4, _CH, emb), lambda i: (i, 0, 0, 0)),
           pl.BlockSpec((blk, 2), lambda i: (i, 0))],
        out_specs=pl.BlockSpec((blk, hid), lambda i: (i, 0)),
        out_shape=jax.ShapeDtypeStruct((n, hid), _F32),
    )(*weights, gath, nm)
    return out.reshape(b, l, hid)


# speedup vs baseline: 2.4852x; 2.4852x over previous
import functools

import jax
import jax.numpy as jnp
from jax import lax
from jax.experimental import pallas as pl
from jax.experimental.pallas import tpu as pltpu
from jax.experimental.pallas import tpu_sc as plsc


_F32 = jnp.float32
_I32 = jnp.int32

_NUM_CORES = 2
_NUM_SUBCORES = 16
_NW = _NUM_CORES * _NUM_SUBCORES
_CH = 256
_LANES = 16


def _prologue_body(bn_ref, ts_ref, el_ref, pid_ref,
                   i1_ref, i2_ref, i3_ref, n0_ref, n1_ref):
    t = ts_ref[...]
    rows, cols = t.shape
    lanes = lax.broadcasted_iota(_I32, (rows, cols), 1)
    tp = jnp.where(lanes >= 1, jnp.roll(t, 1, axis=1), t)
    m = jnp.where(t != tp, tp, -1.0)
    k = 1
    while k < cols:
        m = jnp.maximum(m, jnp.where(lanes >= k, jnp.roll(m, k, axis=1), -1.0))
        k *= 2
    prev = jnp.where(m < 0.0, t, m)
    lag = jnp.clip((t - prev) / 60000.0, 0.0, 1440.0)
    lag_cat = jnp.where(
        lag < 6.0, lag.astype(_I32), ((lag - 1.0) / 10.0).astype(_I32) + 6
    )
    e = el_ref[...]
    el_cat = jnp.clip(e.astype(_I32) + 1, 0, 300)
    i1_ref[...] = lag_cat
    i2_ref[...] = el_cat + 151
    i3_ref[...] = pid_ref[...] + 453
    e_num = jnp.clip(e, 0.0, 300.0)
    lf = jnp.log1p(lag)
    s0 = jnp.sqrt(bn_ref[1, 0] + 1e-5)
    s1 = jnp.sqrt(bn_ref[1, 1] + 1e-5)
    n0_ref[...] = (lf - bn_ref[0, 0]) / s0 * bn_ref[2, 0] + bn_ref[3, 0]
    n1_ref[...] = (e_num - bn_ref[0, 1]) / s1 * bn_ref[2, 1] + bn_ref[3, 1]


def _sc_gather(idx_all, fused_t):
    nch, ntab, ch = idx_all.shape
    vocab, emb = fused_t.shape
    ch_per_w = nch // _NW
    groups = ch // _LANES
    mesh = plsc.VectorSubcoreMesh(
        core_axis_name="c", subcore_axis_name="s",
        num_cores=_NUM_CORES, num_subcores=_NUM_SUBCORES,
    )
    out_type = jax.ShapeDtypeStruct((nch, ntab, emb, ch), _F32)
    scratch = [
        pltpu.VMEM((vocab, emb), _F32),
        pltpu.VMEM((ntab, ch), _I32),
        pltpu.VMEM((ntab, emb, ch), _F32),
        pltpu.SemaphoreType.DMA,
    ]

    @functools.partial(pl.kernel, mesh=mesh, out_type=out_type,
                       scratch_types=scratch,
                       compiler_params=pltpu.CompilerParams(
                           use_tc_tiling_on_sc=False,
                           needs_layout_passes=False))
    def body(idx_h, ft_h, out_h, table_v, idx_v, buf_v, sem):
        wid = lax.axis_index("s") * _NUM_CORES + lax.axis_index("c")
        base = wid * ch_per_w
        pltpu.sync_copy(ft_h, table_v)

        def chunk(j, carry):
            pltpu.sync_copy(idx_h.at[base + j], idx_v)

            def group(g, carry2):
                for t in range(ntab):
                    rows = idx_v[t, pl.ds(g * _LANES, _LANES)]
                    for c in range(emb):
                        col = jnp.full((_LANES,), c, _I32)
                        v = plsc.load_gather(table_v, [rows, col])
                        buf_v[t, c, pl.ds(g * _LANES, _LANES)] = v
                return carry2

            lax.fori_loop(0, groups, group, 0)
            pltpu.sync_copy(buf_v, out_h.at[base + j])
            return carry

        lax.fori_loop(0, ch_per_w, chunk, 0)

    return body(idx_all, fused_t)


def _epilogue_body(rt_ref, wr_ref, wn_ref, wl_ref, we_ref, wp_ref, nw_ref,
                   nb_ref, lb_ref, g_ref, bb_ref, gath_ref, nm_ref, rid_ref,
                   out_ref):
    blk = out_ref.shape[0]
    tdot = lambda xt, w: lax.dot_general(
        xt, w, (((0,), (0,)), ((), ())), preferred_element_type=_F32)
    numemb = (nm_ref[:, 0:1] * nw_ref[0:1, :]
              + nm_ref[:, 1:2] * nw_ref[1:2, :] + nb_ref[...])
    oh = (rid_ref[...] == lax.broadcasted_iota(_I32, (blk, 4), 1)).astype(_F32)
    resp = jnp.dot(oh, rt_ref[...], preferred_element_type=_F32)
    y = jnp.dot(resp, wr_ref[...], preferred_element_type=_F32)
    y = y + jnp.dot(numemb, wn_ref[...], preferred_element_type=_F32)
    y = y + tdot(gath_ref[0, 0], wl_ref[...])
    y = y + tdot(gath_ref[0, 1], we_ref[...])
    y = y + tdot(gath_ref[0, 2], wp_ref[...])
    y = y + lb_ref[...]
    mu = jnp.mean(y, axis=1, keepdims=True)
    d = y - mu
    var = jnp.mean(d * d, axis=1, keepdims=True)
    out_ref[...] = d / jnp.sqrt(var + 1e-12) * g_ref[...] + bb_ref[...]


def kernel(input_ids, position_ids, timestamp, elapsed_time, response_table,
           num_W, num_b, bn_gamma, bn_beta, bn_mean, bn_var, elapsed_table,
           lag_table, pos_table, lin_W, lin_b, ln_gamma, ln_beta):
    b, l = input_ids.shape
    n = b * l
    hid = lin_W.shape[1]
    resp_w = response_table.shape[1]
    emb = lag_table.shape[1]
    nch = n // _CH

    ts_f = timestamp.astype(_F32)
    bn = jnp.stack([bn_mean.astype(_F32), bn_var.astype(_F32),
                    bn_gamma.astype(_F32), bn_beta.astype(_F32)], axis=0)
    rb = 256
    bspec = pl.BlockSpec((rb, l), lambda i: (i, 0))
    i1, i2, i3, n0, n1 = pl.pallas_call(
        _prologue_body,
        grid=(b // rb,),
        in_specs=[pl.BlockSpec((4, 2), lambda i: (0, 0))] + [bspec] * 3,
        out_specs=[bspec] * 5,
        out_shape=[jax.ShapeDtypeStruct((b, l), _I32)] * 3
        + [jax.ShapeDtypeStruct((b, l), _F32)] * 2,
    )(bn, ts_f, elapsed_time.astype(_F32), position_ids.astype(_I32))

    idx_all = jnp.stack(
        [x.reshape(nch, _CH) for x in (i1, i2, i3)], axis=1)
    fused_t = jnp.concatenate([
        lag_table.astype(_F32), elapsed_table.astype(_F32),
        pos_table.astype(_F32)], axis=0)
    gath = _sc_gather(idx_all, fused_t)

    nm = jnp.stack([n0.reshape(n), n1.reshape(n)], axis=-1)
    rid = input_ids.astype(_I32).reshape(n, 1)
    wr = lin_W[0:resp_w]
    wn = lin_W[resp_w:resp_w + emb]
    wl = lin_W[resp_w + emb:resp_w + 2 * emb]
    we = lin_W[resp_w + 2 * emb:resp_w + 3 * emb]
    wp = lin_W[resp_w + 3 * emb:resp_w + 4 * emb]

    blk = _CH
    const = lambda shape: pl.BlockSpec(shape, lambda i: (0, 0))
    weights = (response_table.astype(_F32), wr, wn, wl, we, wp,
               num_W.astype(_F32), num_b.reshape(1, emb),
               lin_b.reshape(1, hid), ln_gamma.reshape(1, hid),
               ln_beta.reshape(1, hid))
    out = pl.pallas_call(
        _epilogue_body,
        grid=(n // blk,),
        in_specs=[const(w.shape) for w in weights]
        + [pl.BlockSpec((1, 3, emb, _CH), lambda i: (i, 0, 0, 0)),
           pl.BlockSpec((blk, 2), lambda i: (i, 0)),
           pl.BlockSpec((blk, 1), lambda i: (i, 0))],
        out_specs=pl.BlockSpec((blk, hid), lambda i: (i, 0)),
        out_shape=jax.ShapeDtypeStruct((n, hid), _F32),
    )(*weights, gath, nm, rid)
    return out.reshape(b, l, hid)


# speedup vs baseline: 3.2940x; 1.3254x over previous
import functools

import jax
import jax.numpy as jnp
from jax import lax
from jax.experimental import pallas as pl
from jax.experimental.pallas import tpu as pltpu
from jax.experimental.pallas import tpu_sc as plsc


_F32 = jnp.float32
_I32 = jnp.int32

_NUM_CORES = 2
_NUM_SUBCORES = 16
_NW = _NUM_CORES * _NUM_SUBCORES
_CH = 256
_LANES = 16


def _prologue_body(bn_ref, ts_ref, el_ref, pid_ref,
                   i1_ref, i2_ref, i3_ref, n0_ref, n1_ref):
    t = ts_ref[...]
    rows, cols = t.shape
    lanes = lax.broadcasted_iota(_I32, (rows, cols), 1)
    tp = jnp.where(lanes >= 1, jnp.roll(t, 1, axis=1), t)
    m = jnp.where(t != tp, tp, -1.0)
    k = 1
    while k < cols:
        m = jnp.maximum(m, jnp.where(lanes >= k, jnp.roll(m, k, axis=1), -1.0))
        k *= 2
    prev = jnp.where(m < 0.0, t, m)
    lag = jnp.clip((t - prev) / 60000.0, 0.0, 1440.0)
    lag_cat = jnp.where(
        lag < 6.0, lag.astype(_I32), ((lag - 1.0) / 10.0).astype(_I32) + 6
    )
    e = el_ref[...]
    el_cat = jnp.clip(e.astype(_I32) + 1, 0, 300)
    i1_ref[...] = lag_cat
    i2_ref[...] = el_cat + 151
    i3_ref[...] = pid_ref[...] + 453
    e_num = jnp.clip(e, 0.0, 300.0)
    lf = jnp.log1p(lag)
    s0 = jnp.sqrt(bn_ref[1, 0] + 1e-5)
    s1 = jnp.sqrt(bn_ref[1, 1] + 1e-5)
    n0_ref[...] = (lf - bn_ref[0, 0]) / s0 * bn_ref[2, 0] + bn_ref[3, 0]
    n1_ref[...] = (e_num - bn_ref[0, 1]) / s1 * bn_ref[2, 1] + bn_ref[3, 1]


def _sc_gather(idx_all, fused_t):
    nch, ntab, ch = idx_all.shape
    vocab, emb_pad = fused_t.shape
    emb = emb_pad - 1
    ch_per_w = nch // _NW
    groups = ch // _LANES
    mesh = plsc.VectorSubcoreMesh(
        core_axis_name="c", subcore_axis_name="s",
        num_cores=_NUM_CORES, num_subcores=_NUM_SUBCORES,
    )
    out_type = jax.ShapeDtypeStruct((nch, ntab, emb, ch), _F32)
    scratch = [
        pltpu.VMEM((vocab, emb_pad), _F32),
        pltpu.VMEM((ntab, ch), _I32),
        pltpu.VMEM((ntab, emb, ch), _F32),
        pltpu.SemaphoreType.DMA,
    ]

    @functools.partial(pl.kernel, mesh=mesh, out_type=out_type,
                       scratch_types=scratch,
                       compiler_params=pltpu.CompilerParams(
                           use_tc_tiling_on_sc=False,
                           needs_layout_passes=False))
    def body(idx_h, ft_h, out_h, table_v, idx_v, buf_v, sem):
        wid = lax.axis_index("s") * _NUM_CORES + lax.axis_index("c")
        base = wid * ch_per_w
        pltpu.sync_copy(ft_h, table_v)

        def chunk(j, carry):
            pltpu.sync_copy(idx_h.at[base + j], idx_v)

            def group(g, carry2):
                for t in range(ntab):
                    rows = idx_v[t, pl.ds(g * _LANES, _LANES)]
                    for c in range(emb):
                        col = jnp.full((_LANES,), c, _I32)
                        v = plsc.load_gather(table_v, [rows, col])
                        buf_v[t, c, pl.ds(g * _LANES, _LANES)] = v
                return carry2

            lax.fori_loop(0, groups, group, 0)
            pltpu.sync_copy(buf_v, out_h.at[base + j])
            return carry

        lax.fori_loop(0, ch_per_w, chunk, 0)

    return body(idx_all, fused_t)


def _epilogue_body(rt_ref, wr_ref, wn_ref, wl_ref, we_ref, wp_ref, nw_ref,
                   nb_ref, lb_ref, g_ref, bb_ref, gath_ref, nm_ref, rid_ref,
                   out_ref):
    blk = out_ref.shape[0]
    tdot = lambda xt, w: lax.dot_general(
        xt, w, (((0,), (0,)), ((), ())), preferred_element_type=_F32)
    numemb = (nm_ref[:, 0:1] * nw_ref[0:1, :]
              + nm_ref[:, 1:2] * nw_ref[1:2, :] + nb_ref[...])
    oh = (rid_ref[...] == lax.broadcasted_iota(_I32, (blk, 4), 1)).astype(_F32)
    resp = jnp.dot(oh, rt_ref[...], preferred_element_type=_F32)
    y = jnp.dot(resp, wr_ref[...], preferred_element_type=_F32)
    y = y + jnp.dot(numemb, wn_ref[...], preferred_element_type=_F32)
    y = y + tdot(gath_ref[0, 0], wl_ref[...])
    y = y + tdot(gath_ref[0, 1], we_ref[...])
    y = y + tdot(gath_ref[0, 2], wp_ref[...])
    y = y + lb_ref[...]
    mu = jnp.mean(y, axis=1, keepdims=True)
    d = y - mu
    var = jnp.mean(d * d, axis=1, keepdims=True)
    out_ref[...] = d / jnp.sqrt(var + 1e-12) * g_ref[...] + bb_ref[...]


def kernel(input_ids, position_ids, timestamp, elapsed_time, response_table,
           num_W, num_b, bn_gamma, bn_beta, bn_mean, bn_var, elapsed_table,
           lag_table, pos_table, lin_W, lin_b, ln_gamma, ln_beta):
    b, l = input_ids.shape
    n = b * l
    hid = lin_W.shape[1]
    resp_w = response_table.shape[1]
    emb = lag_table.shape[1]
    nch = n // _CH

    ts_f = timestamp.astype(_F32)
    bn = jnp.stack([bn_mean.astype(_F32), bn_var.astype(_F32),
                    bn_gamma.astype(_F32), bn_beta.astype(_F32)], axis=0)
    rb = 256
    bspec = pl.BlockSpec((rb, l), lambda i: (i, 0))
    i1, i2, i3, n0, n1 = pl.pallas_call(
        _prologue_body,
        grid=(b // rb,),
        in_specs=[pl.BlockSpec((4, 2), lambda i: (0, 0))] + [bspec] * 3,
        out_specs=[bspec] * 5,
        out_shape=[jax.ShapeDtypeStruct((b, l), _I32)] * 3
        + [jax.ShapeDtypeStruct((b, l), _F32)] * 2,
    )(bn, ts_f, elapsed_time.astype(_F32), position_ids.astype(_I32))

    idx_all = jnp.stack(
        [x.reshape(nch, _CH) for x in (i1, i2, i3)], axis=1)
    fused_t = jnp.pad(jnp.concatenate([
        lag_table.astype(_F32), elapsed_table.astype(_F32),
        pos_table.astype(_F32)], axis=0), ((0, 0), (0, 1)))
    gath = _sc_gather(idx_all, fused_t)

    nm = jnp.stack([n0.reshape(n), n1.reshape(n)], axis=-1)
    rid = input_ids.astype(_I32).reshape(n, 1)
    wr = lin_W[0:resp_w]
    wn = lin_W[resp_w:resp_w + emb]
    wl = lin_W[resp_w + emb:resp_w + 2 * emb]
    we = lin_W[resp_w + 2 * emb:resp_w + 3 * emb]
    wp = lin_W[resp_w + 3 * emb:resp_w + 4 * emb]

    blk = _CH
    const = lambda shape: pl.BlockSpec(shape, lambda i: (0, 0))
    weights = (response_table.astype(_F32), wr, wn, wl, we, wp,
               num_W.astype(_F32), num_b.reshape(1, emb),
               lin_b.reshape(1, hid), ln_gamma.reshape(1, hid),
               ln_beta.reshape(1, hid))
    out = pl.pallas_call(
        _epilogue_body,
        grid=(n // blk,),
        in_specs=[const(w.shape) for w in weights]
        + [pl.BlockSpec((1, 3, emb, _CH), lambda i: (i, 0, 0, 0)),
           pl.BlockSpec((blk, 2), lambda i: (i, 0)),
           pl.BlockSpec((blk, 1), lambda i: (i, 0))],
        out_specs=pl.BlockSpec((blk, hid), lambda i: (i, 0)),
        out_shape=jax.ShapeDtypeStruct((n, hid), _F32),
        compiler_params=pltpu.CompilerParams(
            fuse_transposed_lhs_in_matmul=True),
    )(*weights, gath, nm, rid)
    return out.reshape(b, l, hid)


# speedup vs baseline: 5.2290x; 1.5874x over previous
import functools

import jax
import jax.numpy as jnp
from jax import lax
from jax.experimental import pallas as pl
from jax.experimental.pallas import tpu as pltpu
from jax.experimental.pallas import tpu_sc as plsc


_F32 = jnp.float32
_I32 = jnp.int32

_NUM_CORES = 2
_NUM_SUBCORES = 16
_NW = _NUM_CORES * _NUM_SUBCORES
_CH = 128
_LANES = 16
_NTAB = 3


def _prologue_body(bn_ref, ts_ref, el_ref, pid_ref,
                   i1_ref, i2_ref, i3_ref, n0_ref, n1_ref):
    t = ts_ref[...]
    rows, cols = t.shape
    lanes = lax.broadcasted_iota(_I32, (rows, cols), 1)
    tp = jnp.where(lanes >= 1, jnp.roll(t, 1, axis=1), t)
    m = jnp.where(t != tp, tp, -1.0)
    k = 1
    while k < cols:
        m = jnp.maximum(m, jnp.where(lanes >= k, jnp.roll(m, k, axis=1), -1.0))
        k *= 2
    prev = jnp.where(m < 0.0, t, m)
    lag = jnp.clip((t - prev) / 60000.0, 0.0, 1440.0)
    lag_cat = jnp.where(
        lag < 6.0, lag.astype(_I32), ((lag - 1.0) / 10.0).astype(_I32) + 6
    )
    e = el_ref[...]
    el_cat = jnp.clip(e.astype(_I32) + 1, 0, 300)
    i1_ref[...] = lag_cat
    i2_ref[...] = el_cat + 151
    i3_ref[...] = pid_ref[...] + 453
    e_num = jnp.clip(e, 0.0, 300.0)
    lf = jnp.log1p(lag)
    s0 = jnp.sqrt(bn_ref[1, 0] + 1e-5)
    s1 = jnp.sqrt(bn_ref[1, 1] + 1e-5)
    n0_ref[...] = (lf - bn_ref[0, 0]) / s0 * bn_ref[2, 0] + bn_ref[3, 0]
    n1_ref[...] = (e_num - bn_ref[0, 1]) / s1 * bn_ref[2, 1] + bn_ref[3, 1]


def _sc_gather(idx2, fused_t):
    nrow = idx2.shape[0]
    nch = nrow // _NTAB
    vocab, emb_pad = fused_t.shape
    emb = emb_pad - 1
    seg = _NTAB * emb
    ch_per_w = nch // _NW
    groups = _CH // _LANES
    mesh = plsc.VectorSubcoreMesh(
        core_axis_name="c", subcore_axis_name="s",
        num_cores=_NUM_CORES, num_subcores=_NUM_SUBCORES,
    )
    out_type = jax.ShapeDtypeStruct((nch * seg, _CH), _F32)
    scratch = [
        pltpu.VMEM((vocab, emb_pad), _F32),
        pltpu.VMEM((_NTAB, _CH), _I32),
        pltpu.VMEM((_NTAB, _CH), _I32),
        pltpu.VMEM((_NTAB * emb, _CH), _F32),
        pltpu.VMEM((_NTAB * emb, _CH), _F32),
        pltpu.SemaphoreType.DMA,
        pltpu.SemaphoreType.DMA,
    ]

    @functools.partial(pl.kernel, mesh=mesh, out_type=out_type,
                       scratch_types=scratch,
                       compiler_params=pltpu.CompilerParams(
                           use_tc_tiling_on_sc=False,
                           needs_layout_passes=False))
    def body(idx_h, ft_h, out_h, table_v, ia, ib, ba, bb, sa, sb):
        wid = lax.axis_index("s") * _NUM_CORES + lax.axis_index("c")
        base = wid * ch_per_w
        pltpu.sync_copy(ft_h, table_v)

        def sub(j, ixv, bufv, sem, guard):
            @pl.when(guard)
            def _():
                pltpu.make_async_copy(
                    bufv, out_h.at[pl.ds(0, seg)], sem).wait()

            pltpu.sync_copy(idx_h.at[pl.ds(j * _NTAB, _NTAB)], ixv)

            def group(g, carry):
                for t in range(_NTAB):
                    rows = ixv[t, pl.ds(g * _LANES, _LANES)]
                    for c in range(emb):
                        col = jnp.full((_LANES,), c, _I32)
                        v = plsc.load_gather(table_v, [rows, col])
                        bufv[t * emb + c, pl.ds(g * _LANES, _LANES)] = v
                return carry

            lax.fori_loop(0, groups, group, 0)
            pltpu.async_copy(bufv, out_h.at[pl.ds(j * seg, seg)], sem)

        def pair(k, carry):
            j0 = base + 2 * k
            sub(j0, ia, ba, sa, k >= 1)
            sub(j0 + 1, ib, bb, sb, k >= 1)
            return carry

        lax.fori_loop(0, ch_per_w // 2, pair, 0)
        pltpu.make_async_copy(ba, out_h.at[pl.ds(0, seg)], sa).wait()
        pltpu.make_async_copy(bb, out_h.at[pl.ds(0, seg)], sb).wait()

    return body(idx2, fused_t)


def _epilogue_body(ws_ref, rt_ref, wr_ref, nw2_ref, wn_ref, nb_ref,
                   lb_ref, g_ref, bb_ref, xg_ref, rid_ref, n0_ref, n1_ref,
                   out_ref):
    seg = ws_ref.shape[0]
    kb = rid_ref.shape[1]
    tdot = lambda xt, w: lax.dot_general(
        xt, w, (((0,), (0,)), ((), ())), preferred_element_type=_F32)
    rw = jnp.dot(rt_ref[...], wr_ref[...], preferred_element_type=_F32)
    nw = jnp.dot(nw2_ref[...], wn_ref[...], preferred_element_type=_F32)
    brow = (jnp.dot(nb_ref[...], wn_ref[...], preferred_element_type=_F32)
            + lb_ref[...])
    for q in range(kb):
        x = xg_ref[q * seg:(q + 1) * seg, :]
        y = tdot(x, ws_ref[...])
        oht = (rid_ref[0, q:q + 1, :]
               == lax.broadcasted_iota(_I32, (4, _CH), 0)).astype(_F32)
        y = y + tdot(oht, rw)
        y = y + tdot(n0_ref[0, q:q + 1, :], nw[0:1, :])
        y = y + tdot(n1_ref[0, q:q + 1, :], nw[1:2, :])
        y = y + brow
        mu = jnp.mean(y, axis=1, keepdims=True)
        d = y - mu
        var = jnp.mean(d * d, axis=1, keepdims=True)
        out_ref[q * _CH:(q + 1) * _CH, :] = (
            d / jnp.sqrt(var + 1e-12) * g_ref[...] + bb_ref[...])


def kernel(input_ids, position_ids, timestamp, elapsed_time, response_table,
           num_W, num_b, bn_gamma, bn_beta, bn_mean, bn_var, elapsed_table,
           lag_table, pos_table, lin_W, lin_b, ln_gamma, ln_beta):
    b, l = input_ids.shape
    n = b * l
    hid = lin_W.shape[1]
    resp_w = response_table.shape[1]
    emb = lag_table.shape[1]
    nch = n // _CH
    seg = _NTAB * emb

    ts_f = timestamp.astype(_F32)
    bn = jnp.stack([bn_mean.astype(_F32), bn_var.astype(_F32),
                    bn_gamma.astype(_F32), bn_beta.astype(_F32)], axis=0)
    rb = 256
    bspec = pl.BlockSpec((rb, l), lambda i: (i, 0))
    i1, i2, i3, n0, n1 = pl.pallas_call(
        _prologue_body,
        grid=(b // rb,),
        in_specs=[pl.BlockSpec((4, 2), lambda i: (0, 0))] + [bspec] * 3,
        out_specs=[bspec] * 5,
        out_shape=[jax.ShapeDtypeStruct((b, l), _I32)] * 3
        + [jax.ShapeDtypeStruct((b, l), _F32)] * 2,
    )(bn, ts_f, elapsed_time.astype(_F32), position_ids.astype(_I32))

    idx2 = jnp.stack([x.reshape(nch, _CH) for x in (i1, i2, i3)],
                     axis=1).reshape(nch * _NTAB, _CH)
    fused_t = jnp.pad(jnp.concatenate([
        lag_table.astype(_F32), elapsed_table.astype(_F32),
        pos_table.astype(_F32)], axis=0), ((0, 0), (0, 1)))
    gath = _sc_gather(idx2, fused_t)

    kb = 4
    ridt = input_ids.astype(_I32).reshape(nch // kb, kb, _CH)
    n0t = n0.reshape(nch // kb, kb, _CH)
    n1t = n1.reshape(nch // kb, kb, _CH)
    wr = lin_W[0:resp_w]
    wn = lin_W[resp_w:resp_w + emb]
    wstack = lin_W[resp_w + emb:resp_w + 4 * emb]

    const = lambda shape: pl.BlockSpec(shape, lambda i: (0, 0))
    weights = (wstack, response_table.astype(_F32), wr, num_W.astype(_F32),
               wn, num_b.reshape(1, emb), lin_b.reshape(1, hid),
               ln_gamma.reshape(1, hid), ln_beta.reshape(1, hid))
    out = pl.pallas_call(
        _epilogue_body,
        grid=(nch // kb,),
        in_specs=[const(w.shape) for w in weights]
        + [pl.BlockSpec((kb * seg, _CH), lambda i: (i, 0)),
           pl.BlockSpec((1, kb, _CH), lambda i: (i, 0, 0)),
           pl.BlockSpec((1, kb, _CH), lambda i: (i, 0, 0)),
           pl.BlockSpec((1, kb, _CH), lambda i: (i, 0, 0))],
        out_specs=pl.BlockSpec((kb * _CH, hid), lambda i: (i, 0)),
        out_shape=jax.ShapeDtypeStruct((n, hid), _F32),
        compiler_params=pltpu.CompilerParams(
            fuse_transposed_lhs_in_matmul=True),
    )(*weights, gath, ridt, n0t, n1t)
    return out.reshape(b, l, hid)


# speedup vs baseline: 6.7025x; 1.2818x over previous
import functools

import jax
import jax.numpy as jnp
from jax import lax
from jax.experimental import pallas as pl
from jax.experimental.pallas import tpu as pltpu
from jax.experimental.pallas import tpu_sc as plsc


_F32 = jnp.float32
_I32 = jnp.int32

_NUM_CORES = 2
_NUM_SUBCORES = 16
_NW = _NUM_CORES * _NUM_SUBCORES
_CH = 128
_LANES = 16
_NTAB = 3


def _prologue_body(bn_ref, ts_ref, el_ref, pid_ref,
                   i1_ref, i2_ref, i3_ref, n0_ref, n1_ref):
    t = ts_ref[...]
    rows, cols = t.shape
    lanes = lax.broadcasted_iota(_I32, (rows, cols), 1)
    tp = jnp.where(lanes >= 1, jnp.roll(t, 1, axis=1), t)
    m = jnp.where(t != tp, tp, -1.0)
    k = 1
    while k < cols:
        m = jnp.maximum(m, jnp.where(lanes >= k, jnp.roll(m, k, axis=1), -1.0))
        k *= 2
    prev = jnp.where(m < 0.0, t, m)
    lag = jnp.clip((t - prev) / 60000.0, 0.0, 1440.0)
    lag_cat = jnp.where(
        lag < 6.0, lag.astype(_I32), ((lag - 1.0) / 10.0).astype(_I32) + 6
    )
    e = el_ref[...]
    el_cat = jnp.clip(e.astype(_I32) + 1, 0, 300)
    i1_ref[...] = lag_cat
    i2_ref[...] = el_cat + 151
    i3_ref[...] = pid_ref[...] + 453
    e_num = jnp.clip(e, 0.0, 300.0)
    lf = jnp.log1p(lag)
    s0 = jnp.sqrt(bn_ref[1, 0] + 1e-5)
    s1 = jnp.sqrt(bn_ref[1, 1] + 1e-5)
    n0_ref[...] = (lf - bn_ref[0, 0]) / s0 * bn_ref[2, 0] + bn_ref[3, 0]
    n1_ref[...] = (e_num - bn_ref[0, 1]) / s1 * bn_ref[2, 1] + bn_ref[3, 1]


def _sc_gather(idx2, fused_flat):
    nrow = idx2.shape[0]
    nch = nrow // _NTAB
    emb = 64
    emb_pad = emb + 1
    vocab = fused_flat.shape[0] // emb_pad
    seg = _NTAB * emb
    ch_per_w = nch // _NW
    groups = _CH // _LANES
    mesh = plsc.VectorSubcoreMesh(
        core_axis_name="c", subcore_axis_name="s",
        num_cores=_NUM_CORES, num_subcores=_NUM_SUBCORES,
    )
    out_type = jax.ShapeDtypeStruct((nch * seg, _CH), _F32)
    scratch = [
        pltpu.VMEM((vocab * emb_pad,), _F32),
        pltpu.VMEM((_NTAB, _CH), _I32),
        pltpu.VMEM((_NTAB, _CH), _I32),
        pltpu.VMEM((_NTAB * emb, _CH), _F32),
        pltpu.VMEM((_NTAB * emb, _CH), _F32),
        pltpu.SemaphoreType.DMA,
        pltpu.SemaphoreType.DMA,
    ]

    @functools.partial(pl.kernel, mesh=mesh, out_type=out_type,
                       scratch_types=scratch,
                       compiler_params=pltpu.CompilerParams(
                           use_tc_tiling_on_sc=False,
                           needs_layout_passes=False))
    def body(idx_h, ft_h, out_h, table_v, ia, ib, ba, bb, sa, sb):
        wid = lax.axis_index("s") * _NUM_CORES + lax.axis_index("c")
        base = wid * ch_per_w
        pltpu.sync_copy(ft_h, table_v)

        def sub(j, ixv, bufv, sem, guard):
            @pl.when(guard)
            def _():
                pltpu.make_async_copy(
                    bufv, out_h.at[pl.ds(0, seg)], sem).wait()

            pltpu.sync_copy(idx_h.at[pl.ds(j * _NTAB, _NTAB)], ixv)

            def group(g, carry):
                for t in range(_NTAB):
                    rows = ixv[t, pl.ds(g * _LANES, _LANES)]
                    rs = rows * emb_pad
                    for c in range(emb):
                        v = plsc.load_gather(table_v, [rs + c])
                        bufv[t * emb + c, pl.ds(g * _LANES, _LANES)] = v
                return carry

            lax.fori_loop(0, groups, group, 0)
            pltpu.async_copy(bufv, out_h.at[pl.ds(j * seg, seg)], sem)

        def pair(k, carry):
            j0 = base + 2 * k
            sub(j0, ia, ba, sa, k >= 1)
            sub(j0 + 1, ib, bb, sb, k >= 1)
            return carry

        lax.fori_loop(0, ch_per_w // 2, pair, 0)
        pltpu.make_async_copy(ba, out_h.at[pl.ds(0, seg)], sa).wait()
        pltpu.make_async_copy(bb, out_h.at[pl.ds(0, seg)], sb).wait()

    return body(idx2, fused_flat)


def _epilogue_body(ws_ref, rt_ref, wr_ref, nw2_ref, wn_ref, nb_ref,
                   lb_ref, g_ref, bb_ref, xg_ref, rid_ref, n0_ref, n1_ref,
                   out_ref):
    seg = ws_ref.shape[0]
    kb = rid_ref.shape[1]
    tdot = lambda xt, w: lax.dot_general(
        xt, w, (((0,), (0,)), ((), ())), preferred_element_type=_F32)
    rw = jnp.dot(rt_ref[...], wr_ref[...], preferred_element_type=_F32)
    nw = jnp.dot(nw2_ref[...], wn_ref[...], preferred_element_type=_F32)
    brow = (jnp.dot(nb_ref[...], wn_ref[...], preferred_element_type=_F32)
            + lb_ref[...])
    for q in range(kb):
        x = xg_ref[q * seg:(q + 1) * seg, :]
        y = tdot(x, ws_ref[...])
        oht = (rid_ref[0, q:q + 1, :]
               == lax.broadcasted_iota(_I32, (4, _CH), 0)).astype(_F32)
        y = y + tdot(oht, rw)
        y = y + tdot(n0_ref[0, q:q + 1, :], nw[0:1, :])
        y = y + tdot(n1_ref[0, q:q + 1, :], nw[1:2, :])
        y = y + brow
        mu = jnp.mean(y, axis=1, keepdims=True)
        d = y - mu
        var = jnp.mean(d * d, axis=1, keepdims=True)
        out_ref[q * _CH:(q + 1) * _CH, :] = (
            d / jnp.sqrt(var + 1e-12) * g_ref[...] + bb_ref[...])


def kernel(input_ids, position_ids, timestamp, elapsed_time, response_table,
           num_W, num_b, bn_gamma, bn_beta, bn_mean, bn_var, elapsed_table,
           lag_table, pos_table, lin_W, lin_b, ln_gamma, ln_beta):
    b, l = input_ids.shape
    n = b * l
    hid = lin_W.shape[1]
    resp_w = response_table.shape[1]
    emb = lag_table.shape[1]
    nch = n // _CH
    seg = _NTAB * emb

    ts_f = timestamp.astype(_F32)
    bn = jnp.stack([bn_mean.astype(_F32), bn_var.astype(_F32),
                    bn_gamma.astype(_F32), bn_beta.astype(_F32)], axis=0)
    rb = 256
    bspec = pl.BlockSpec((rb, l), lambda i: (i, 0))
    i1, i2, i3, n0, n1 = pl.pallas_call(
        _prologue_body,
        grid=(b // rb,),
        in_specs=[pl.BlockSpec((4, 2), lambda i: (0, 0))] + [bspec] * 3,
        out_specs=[bspec] * 5,
        out_shape=[jax.ShapeDtypeStruct((b, l), _I32)] * 3
        + [jax.ShapeDtypeStruct((b, l), _F32)] * 2,
    )(bn, ts_f, elapsed_time.astype(_F32), position_ids.astype(_I32))

    idx2 = jnp.stack([x.reshape(nch, _CH) for x in (i1, i2, i3)],
                     axis=1).reshape(nch * _NTAB, _CH)
    fused_t = jnp.pad(jnp.concatenate([
        lag_table.astype(_F32), elapsed_table.astype(_F32),
        pos_table.astype(_F32)], axis=0), ((0, 0), (0, 1)))
    gath = _sc_gather(idx2, fused_t.reshape(-1))

    kb = 8
    ridt = input_ids.astype(_I32).reshape(nch // kb, kb, _CH)
    n0t = n0.reshape(nch // kb, kb, _CH)
    n1t = n1.reshape(nch // kb, kb, _CH)
    wr = lin_W[0:resp_w]
    wn = lin_W[resp_w:resp_w + emb]
    wstack = lin_W[resp_w + emb:resp_w + 4 * emb]

    const = lambda shape: pl.BlockSpec(shape, lambda i: (0, 0))
    weights = (wstack, response_table.astype(_F32), wr, num_W.astype(_F32),
               wn, num_b.reshape(1, emb), lin_b.reshape(1, hid),
               ln_gamma.reshape(1, hid), ln_beta.reshape(1, hid))
    out = pl.pallas_call(
        _epilogue_body,
        grid=(nch // kb,),
        in_specs=[const(w.shape) for w in weights]
        + [pl.BlockSpec((kb * seg, _CH), lambda i: (i, 0)),
           pl.BlockSpec((1, kb, _CH), lambda i: (i, 0, 0)),
           pl.BlockSpec((1, kb, _CH), lambda i: (i, 0, 0)),
           pl.BlockSpec((1, kb, _CH), lambda i: (i, 0, 0))],
        out_specs=pl.BlockSpec((kb * _CH, hid), lambda i: (i, 0)),
        out_shape=jax.ShapeDtypeStruct((n, hid), _F32),
        compiler_params=pltpu.CompilerParams(
            fuse_transposed_lhs_in_matmul=True),
    )(*weights, gath, ridt, n0t, n1t)
    return out.reshape(b, l, hid)


# speedup vs baseline: 7.4262x; 1.1080x over previous
import functools

import jax
import jax.numpy as jnp
from jax import lax
from jax.experimental import pallas as pl
from jax.experimental.pallas import tpu as pltpu
from jax.experimental.pallas import tpu_sc as plsc


_F32 = jnp.float32
_I32 = jnp.int32

_NUM_CORES = 2
_NUM_SUBCORES = 16
_NW = _NUM_CORES * _NUM_SUBCORES
_CH = 128
_LANES = 16
_NTAB = 3


def _prologue_body(bn_ref, ts_ref, el_ref, pid_ref,
                   i1_ref, i2_ref, i3_ref, n0_ref, n1_ref):
    t = ts_ref[...]
    rows, cols = t.shape
    lanes = lax.broadcasted_iota(_I32, (rows, cols), 1)
    tp = jnp.where(lanes >= 1, jnp.roll(t, 1, axis=1), t)
    m = jnp.where(t != tp, tp, -1.0)
    k = 1
    while k < cols:
        m = jnp.maximum(m, jnp.where(lanes >= k, jnp.roll(m, k, axis=1), -1.0))
        k *= 2
    prev = jnp.where(m < 0.0, t, m)
    lag = jnp.clip((t - prev) / 60000.0, 0.0, 1440.0)
    lag_cat = jnp.where(
        lag < 6.0, lag.astype(_I32), ((lag - 1.0) / 10.0).astype(_I32) + 6
    )
    e = el_ref[...]
    el_cat = jnp.clip(e.astype(_I32) + 1, 0, 300)
    i1_ref[...] = lag_cat
    i2_ref[...] = el_cat + 151
    i3_ref[...] = pid_ref[...] + 453
    e_num = jnp.clip(e, 0.0, 300.0)
    lf = jnp.log1p(lag)
    s0 = jnp.sqrt(bn_ref[1, 0] + 1e-5)
    s1 = jnp.sqrt(bn_ref[1, 1] + 1e-5)
    n0_ref[...] = (lf - bn_ref[0, 0]) / s0 * bn_ref[2, 0] + bn_ref[3, 0]
    n1_ref[...] = (e_num - bn_ref[0, 1]) / s1 * bn_ref[2, 1] + bn_ref[3, 1]


def _sc_gather(idx2, fused_flat):
    nrow = idx2.shape[0]
    nch = nrow // _NTAB
    emb = 64
    emb_pad = emb + 1
    vocab = fused_flat.shape[0] // emb_pad
    seg = _NTAB * emb
    ch_per_w = nch // _NW
    groups = _CH // _LANES
    mesh = plsc.VectorSubcoreMesh(
        core_axis_name="c", subcore_axis_name="s",
        num_cores=_NUM_CORES, num_subcores=_NUM_SUBCORES,
    )
    out_type = jax.ShapeDtypeStruct((nch * seg, _CH), _F32)
    slab = 10
    scratch = [
        pltpu.VMEM((vocab * emb_pad,), _F32),
        pltpu.VMEM((slab * _NTAB, _CH), _I32),
        pltpu.VMEM((_NTAB * emb, _CH), _F32),
        pltpu.VMEM((_NTAB * emb, _CH), _F32),
        pltpu.SemaphoreType.DMA,
        pltpu.SemaphoreType.DMA,
    ]

    @functools.partial(pl.kernel, mesh=mesh, out_type=out_type,
                       scratch_types=scratch,
                       compiler_params=pltpu.CompilerParams(
                           use_tc_tiling_on_sc=False,
                           needs_layout_passes=False))
    def body(idx_h, ft_h, out_h, table_v, islab, ba, bb, sa, sb):
        wid = lax.axis_index("s") * _NUM_CORES + lax.axis_index("c")
        base = wid * ch_per_w
        pltpu.sync_copy(ft_h, table_v)

        def sub(j, jl, bufv, sem, guard):
            @pl.when(guard)
            def _():
                pltpu.make_async_copy(
                    bufv, out_h.at[pl.ds(0, seg)], sem).wait()

            def group(g, carry):
                for t in range(_NTAB):
                    rows = islab[jl * _NTAB + t, pl.ds(g * _LANES, _LANES)]
                    rs = rows * emb_pad
                    for c in range(emb):
                        v = plsc.load_gather(table_v, [rs + c])
                        bufv[t * emb + c, pl.ds(g * _LANES, _LANES)] = v
                return carry

            lax.fori_loop(0, groups, group, 0)
            pltpu.async_copy(bufv, out_h.at[pl.ds(j * seg, seg)], sem)

        def pair(k, carry):
            j0 = base + 2 * k
            jl0 = lax.rem(2 * k, slab)
            sub(j0, jl0, ba, sa, k >= 1)
            sub(j0 + 1, jl0 + 1, bb, sb, k >= 1)
            return carry

        def slab_loop(h, carry):
            pltpu.sync_copy(
                idx_h.at[pl.ds((base + h * slab) * _NTAB, slab * _NTAB)],
                islab)
            lax.fori_loop(h * slab // 2, (h + 1) * slab // 2, pair, 0)
            return carry

        lax.fori_loop(0, ch_per_w // slab, slab_loop, 0)
        pltpu.make_async_copy(ba, out_h.at[pl.ds(0, seg)], sa).wait()
        pltpu.make_async_copy(bb, out_h.at[pl.ds(0, seg)], sb).wait()

    return body(idx2, fused_flat)


def _epilogue_body(ws_ref, rt_ref, wr_ref, nw2_ref, wn_ref, nb_ref,
                   lb_ref, g_ref, bb_ref, xg_ref, rid_ref, n0_ref, n1_ref,
                   out_ref):
    seg = ws_ref.shape[0]
    kb = rid_ref.shape[1]
    tdot = lambda xt, w: lax.dot_general(
        xt, w, (((0,), (0,)), ((), ())), preferred_element_type=_F32)
    rw = jnp.dot(rt_ref[...], wr_ref[...], preferred_element_type=_F32)
    nw = jnp.dot(nw2_ref[...], wn_ref[...], preferred_element_type=_F32)
    brow = (jnp.dot(nb_ref[...], wn_ref[...], preferred_element_type=_F32)
            + lb_ref[...])
    for q in range(kb):
        x = xg_ref[q * seg:(q + 1) * seg, :]
        y = tdot(x, ws_ref[...])
        oht = (rid_ref[0, q:q + 1, :]
               == lax.broadcasted_iota(_I32, (4, _CH), 0)).astype(_F32)
        y = y + tdot(oht, rw)
        y = y + tdot(n0_ref[0, q:q + 1, :], nw[0:1, :])
        y = y + tdot(n1_ref[0, q:q + 1, :], nw[1:2, :])
        y = y + brow
        mu = jnp.mean(y, axis=1, keepdims=True)
        d = y - mu
        var = jnp.mean(d * d, axis=1, keepdims=True)
        out_ref[q * _CH:(q + 1) * _CH, :] = (
            d / jnp.sqrt(var + 1e-12) * g_ref[...] + bb_ref[...])


def kernel(input_ids, position_ids, timestamp, elapsed_time, response_table,
           num_W, num_b, bn_gamma, bn_beta, bn_mean, bn_var, elapsed_table,
           lag_table, pos_table, lin_W, lin_b, ln_gamma, ln_beta):
    b, l = input_ids.shape
    n = b * l
    hid = lin_W.shape[1]
    resp_w = response_table.shape[1]
    emb = lag_table.shape[1]
    nch = n // _CH
    seg = _NTAB * emb

    ts_f = timestamp.astype(_F32)
    bn = jnp.stack([bn_mean.astype(_F32), bn_var.astype(_F32),
                    bn_gamma.astype(_F32), bn_beta.astype(_F32)], axis=0)
    rb = 256
    bspec = pl.BlockSpec((rb, l), lambda i: (i, 0))
    i1, i2, i3, n0, n1 = pl.pallas_call(
        _prologue_body,
        grid=(b // rb,),
        in_specs=[pl.BlockSpec((4, 2), lambda i: (0, 0))] + [bspec] * 3,
        out_specs=[bspec] * 5,
        out_shape=[jax.ShapeDtypeStruct((b, l), _I32)] * 3
        + [jax.ShapeDtypeStruct((b, l), _F32)] * 2,
    )(bn, ts_f, elapsed_time.astype(_F32), position_ids.astype(_I32))

    idx2 = jnp.stack([x.reshape(nch, _CH) for x in (i1, i2, i3)],
                     axis=1).reshape(nch * _NTAB, _CH)
    fused_t = jnp.pad(jnp.concatenate([
        lag_table.astype(_F32), elapsed_table.astype(_F32),
        pos_table.astype(_F32)], axis=0), ((0, 0), (0, 1)))
    gath = _sc_gather(idx2, fused_t.reshape(-1))

    kb = 16
    ridt = input_ids.astype(_I32).reshape(nch // kb, kb, _CH)
    n0t = n0.reshape(nch // kb, kb, _CH)
    n1t = n1.reshape(nch // kb, kb, _CH)
    wr = lin_W[0:resp_w]
    wn = lin_W[resp_w:resp_w + emb]
    wstack = lin_W[resp_w + emb:resp_w + 4 * emb]

    const = lambda shape: pl.BlockSpec(shape, lambda i: (0, 0))
    weights = (wstack, response_table.astype(_F32), wr, num_W.astype(_F32),
               wn, num_b.reshape(1, emb), lin_b.reshape(1, hid),
               ln_gamma.reshape(1, hid), ln_beta.reshape(1, hid))
    out = pl.pallas_call(
        _epilogue_body,
        grid=(nch // kb,),
        in_specs=[const(w.shape) for w in weights]
        + [pl.BlockSpec((kb * seg, _CH), lambda i: (i, 0)),
           pl.BlockSpec((1, kb, _CH), lambda i: (i, 0, 0)),
           pl.BlockSpec((1, kb, _CH), lambda i: (i, 0, 0)),
           pl.BlockSpec((1, kb, _CH), lambda i: (i, 0, 0))],
        out_specs=pl.BlockSpec((kb * _CH, hid), lambda i: (i, 0)),
        out_shape=jax.ShapeDtypeStruct((n, hid), _F32),
        compiler_params=pltpu.CompilerParams(
            fuse_transposed_lhs_in_matmul=True),
    )(*weights, gath, ridt, n0t, n1t)
    return out.reshape(b, l, hid)


# speedup vs baseline: 7.9054x; 1.0645x over previous
import functools

import jax
import jax.numpy as jnp
from jax import lax
from jax.experimental import pallas as pl
from jax.experimental.pallas import tpu as pltpu
from jax.experimental.pallas import tpu_sc as plsc


_F32 = jnp.float32
_I32 = jnp.int32

_NUM_CORES = 2
_NUM_SUBCORES = 16
_NW = _NUM_CORES * _NUM_SUBCORES
_CH = 128
_LANES = 16
_NTAB = 3


def _prologue_body(bn_ref, ts_ref, el_ref, pid_ref,
                   i1_ref, i2_ref, i3_ref, n0_ref, n1_ref):
    t = ts_ref[...]
    rows, cols = t.shape
    lanes = lax.broadcasted_iota(_I32, (rows, cols), 1)
    tp = jnp.where(lanes >= 1, jnp.roll(t, 1, axis=1), t)
    m = jnp.where(t != tp, tp, -1.0)
    k = 1
    while k < cols:
        m = jnp.maximum(m, jnp.where(lanes >= k, jnp.roll(m, k, axis=1), -1.0))
        k *= 2
    prev = jnp.where(m < 0.0, t, m)
    lag = jnp.clip((t - prev) / 60000.0, 0.0, 1440.0)
    lag_cat = jnp.where(
        lag < 6.0, lag.astype(_I32), ((lag - 1.0) / 10.0).astype(_I32) + 6
    )
    e = el_ref[...]
    el_cat = jnp.clip(e.astype(_I32) + 1, 0, 300)
    i1_ref[...] = lag_cat
    i2_ref[...] = el_cat + 151
    i3_ref[...] = pid_ref[...] + 453
    e_num = jnp.clip(e, 0.0, 300.0)
    lf = jnp.log1p(lag)
    s0 = jnp.sqrt(bn_ref[1, 0] + 1e-5)
    s1 = jnp.sqrt(bn_ref[1, 1] + 1e-5)
    n0_ref[...] = (lf - bn_ref[0, 0]) / s0 * bn_ref[2, 0] + bn_ref[3, 0]
    n1_ref[...] = (e_num - bn_ref[0, 1]) / s1 * bn_ref[2, 1] + bn_ref[3, 1]


def _sc_gather(idx2, fused_flat):
    nrow = idx2.shape[0]
    nch = nrow // _NTAB
    emb = 64
    emb_pad = emb + 1
    vocab = fused_flat.shape[0] // emb_pad
    seg = _NTAB * emb
    ch_per_w = nch // _NW
    groups = _CH // _LANES
    mesh = plsc.VectorSubcoreMesh(
        core_axis_name="c", subcore_axis_name="s",
        num_cores=_NUM_CORES, num_subcores=_NUM_SUBCORES,
    )
    out_type = jax.ShapeDtypeStruct((nch * seg, _CH), _F32)
    slab = ch_per_w
    scratch = [
        pltpu.VMEM((vocab * emb_pad,), _F32),
        pltpu.VMEM((slab * _NTAB, _CH), _I32),
        pltpu.VMEM((_NTAB * emb, _CH), _F32),
        pltpu.VMEM((_NTAB * emb, _CH), _F32),
        pltpu.SemaphoreType.DMA,
        pltpu.SemaphoreType.DMA,
    ]

    @functools.partial(pl.kernel, mesh=mesh, out_type=out_type,
                       scratch_types=scratch,
                       compiler_params=pltpu.CompilerParams(
                           use_tc_tiling_on_sc=False,
                           needs_layout_passes=False))
    def body(idx_h, ft_h, out_h, table_v, islab, ba, bb, sa, sb):
        wid = lax.axis_index("s") * _NUM_CORES + lax.axis_index("c")
        base = wid * ch_per_w
        pltpu.sync_copy(ft_h, table_v)

        def sub(j, jl, bufv, sem, guard):
            @pl.when(guard)
            def _():
                pltpu.make_async_copy(
                    bufv, out_h.at[pl.ds(0, seg)], sem).wait()

            def group(g, carry):
                for t in range(_NTAB):
                    rows = islab[jl * _NTAB + t, pl.ds(g * _LANES, _LANES)]
                    rs = rows * emb_pad
                    for c in range(emb):
                        v = plsc.load_gather(table_v, [rs + c])
                        bufv[t * emb + c, pl.ds(g * _LANES, _LANES)] = v
                return carry

            lax.fori_loop(0, groups, group, 0)
            pltpu.async_copy(bufv, out_h.at[pl.ds(j * seg, seg)], sem)

        def pair(k, carry):
            j0 = base + 2 * k
            jl0 = lax.rem(2 * k, slab)
            sub(j0, jl0, ba, sa, k >= 1)
            sub(j0 + 1, jl0 + 1, bb, sb, k >= 1)
            return carry

        def slab_loop(h, carry):
            pltpu.sync_copy(
                idx_h.at[pl.ds((base + h * slab) * _NTAB, slab * _NTAB)],
                islab)
            lax.fori_loop(h * slab // 2, (h + 1) * slab // 2, pair, 0)
            return carry

        lax.fori_loop(0, ch_per_w // slab, slab_loop, 0)
        pltpu.make_async_copy(ba, out_h.at[pl.ds(0, seg)], sa).wait()
        pltpu.make_async_copy(bb, out_h.at[pl.ds(0, seg)], sb).wait()

    return body(idx2, fused_flat)


def _epilogue_body(ws_ref, rt_ref, wr_ref, nw2_ref, wn_ref, nb_ref,
                   lb_ref, g_ref, bb_ref, xg_ref, rid_ref, n0_ref, n1_ref,
                   out_ref):
    seg = ws_ref.shape[0]
    kb = rid_ref.shape[1]
    tdot = lambda xt, w: lax.dot_general(
        xt, w, (((0,), (0,)), ((), ())), preferred_element_type=_F32)
    rw = jnp.dot(rt_ref[...], wr_ref[...], preferred_element_type=_F32)
    nw = jnp.dot(nw2_ref[...], wn_ref[...], preferred_element_type=_F32)
    brow = (jnp.dot(nb_ref[...], wn_ref[...], preferred_element_type=_F32)
            + lb_ref[...])
    for q in range(kb):
        x = xg_ref[q * seg:(q + 1) * seg, :]
        y = tdot(x, ws_ref[...])
        oht = (rid_ref[0, q:q + 1, :]
               == lax.broadcasted_iota(_I32, (4, _CH), 0)).astype(_F32)
        y = y + tdot(oht, rw)
        y = y + tdot(n0_ref[0, q:q + 1, :], nw[0:1, :])
        y = y + tdot(n1_ref[0, q:q + 1, :], nw[1:2, :])
        y = y + brow
        mu = jnp.mean(y, axis=1, keepdims=True)
        d = y - mu
        var = jnp.mean(d * d, axis=1, keepdims=True)
        out_ref[q * _CH:(q + 1) * _CH, :] = (
            d / jnp.sqrt(var + 1e-12) * g_ref[...] + bb_ref[...])


def kernel(input_ids, position_ids, timestamp, elapsed_time, response_table,
           num_W, num_b, bn_gamma, bn_beta, bn_mean, bn_var, elapsed_table,
           lag_table, pos_table, lin_W, lin_b, ln_gamma, ln_beta):
    b, l = input_ids.shape
    n = b * l
    hid = lin_W.shape[1]
    resp_w = response_table.shape[1]
    emb = lag_table.shape[1]
    nch = n // _CH
    seg = _NTAB * emb

    ts_f = timestamp.astype(_F32)
    bn = jnp.stack([bn_mean.astype(_F32), bn_var.astype(_F32),
                    bn_gamma.astype(_F32), bn_beta.astype(_F32)], axis=0)
    rb = 256
    bspec = pl.BlockSpec((rb, l), lambda i: (i, 0))
    i1, i2, i3, n0, n1 = pl.pallas_call(
        _prologue_body,
        grid=(b // rb,),
        in_specs=[pl.BlockSpec((4, 2), lambda i: (0, 0))] + [bspec] * 3,
        out_specs=[bspec] * 5,
        out_shape=[jax.ShapeDtypeStruct((b, l), _I32)] * 3
        + [jax.ShapeDtypeStruct((b, l), _F32)] * 2,
    )(bn, ts_f, elapsed_time.astype(_F32), position_ids.astype(_I32))

    idx2 = jnp.stack([x.reshape(nch, _CH) for x in (i1, i2, i3)],
                     axis=1).reshape(nch * _NTAB, _CH)
    fused_flat = jnp.pad(jnp.concatenate([
        lag_table.astype(_F32), elapsed_table.astype(_F32),
        pos_table.astype(_F32)], axis=0), ((0, 0), (0, 1))).reshape(-1)

    kb = 16
    ridt = input_ids.astype(_I32).reshape(nch, _CH)
    n0t = n0.reshape(nch, _CH)
    n1t = n1.reshape(nch, _CH)
    wr = lin_W[0:resp_w]
    wn = lin_W[resp_w:resp_w + emb]
    wstack = lin_W[resp_w + emb:resp_w + 4 * emb]

    const = lambda shape: pl.BlockSpec(shape, lambda i: (0, 0))
    weights = (wstack, response_table.astype(_F32), wr, num_W.astype(_F32),
               wn, num_b.reshape(1, emb), lin_b.reshape(1, hid),
               ln_gamma.reshape(1, hid), ln_beta.reshape(1, hid))

    def epilogue(gath_p, ridt_p, n0t_p, n1t_p):
        nch_p = ridt_p.shape[0]
        return pl.pallas_call(
            _epilogue_body,
            grid=(nch_p // kb,),
            in_specs=[const(w.shape) for w in weights]
            + [pl.BlockSpec((kb * seg, _CH), lambda i: (i, 0)),
               pl.BlockSpec((1, kb, _CH), lambda i: (i, 0, 0)),
               pl.BlockSpec((1, kb, _CH), lambda i: (i, 0, 0)),
               pl.BlockSpec((1, kb, _CH), lambda i: (i, 0, 0))],
            out_specs=pl.BlockSpec((kb * _CH, hid), lambda i: (i, 0)),
            out_shape=jax.ShapeDtypeStruct((nch_p * _CH, hid), _F32),
            compiler_params=pltpu.CompilerParams(
                fuse_transposed_lhs_in_matmul=True),
        )(*weights, gath_p,
          ridt_p.reshape(nch_p // kb, kb, _CH),
          n0t_p.reshape(nch_p // kb, kb, _CH),
          n1t_p.reshape(nch_p // kb, kb, _CH))

    ncha = 26 * _NW
    outs = []
    parts = []
    for lo, hi in ((0, ncha), (ncha, nch)):
        parts.append((
            _sc_gather(idx2[lo * _NTAB:hi * _NTAB], fused_flat),
            ridt[lo:hi], n0t[lo:hi], n1t[lo:hi]))
    for p in parts:
        outs.append(epilogue(*p))
    out = jnp.concatenate(outs, axis=0)
    return out.reshape(b, l, hid)


# speedup vs baseline: 8.0950x; 1.0240x over previous
import functools

import jax
import jax.numpy as jnp
from jax import lax
from jax.experimental import pallas as pl
from jax.experimental.pallas import tpu as pltpu
from jax.experimental.pallas import tpu_sc as plsc


_F32 = jnp.float32
_I32 = jnp.int32

_NUM_CORES = 2
_NUM_SUBCORES = 16
_NW = _NUM_CORES * _NUM_SUBCORES
_CH = 128
_LANES = 16
_NTAB = 3


def _prologue_body(bn_ref, ts_ref, el_ref, pid_ref,
                   i1_ref, i2_ref, i3_ref, n0_ref, n1_ref):
    t = ts_ref[...]
    rows, cols = t.shape
    lanes = lax.broadcasted_iota(_I32, (rows, cols), 1)
    tp = jnp.where(lanes >= 1, jnp.roll(t, 1, axis=1), t)
    m = jnp.where(t != tp, tp, -1.0)
    k = 1
    while k < cols:
        m = jnp.maximum(m, jnp.where(lanes >= k, jnp.roll(m, k, axis=1), -1.0))
        k *= 2
    prev = jnp.where(m < 0.0, t, m)
    lag = jnp.clip((t - prev) / 60000.0, 0.0, 1440.0)
    lag_cat = jnp.where(
        lag < 6.0, lag.astype(_I32), ((lag - 1.0) / 10.0).astype(_I32) + 6
    )
    e = el_ref[...]
    el_cat = jnp.clip(e.astype(_I32) + 1, 0, 300)
    i1_ref[...] = lag_cat
    i2_ref[...] = el_cat + 151
    i3_ref[...] = pid_ref[...] + 453
    e_num = jnp.clip(e, 0.0, 300.0)
    lf = jnp.log1p(lag)
    s0 = jnp.sqrt(bn_ref[1, 0] + 1e-5)
    s1 = jnp.sqrt(bn_ref[1, 1] + 1e-5)
    n0_ref[...] = (lf - bn_ref[0, 0]) / s0 * bn_ref[2, 0] + bn_ref[3, 0]
    n1_ref[...] = (e_num - bn_ref[0, 1]) / s1 * bn_ref[2, 1] + bn_ref[3, 1]


def _sc_gather(idx2, fused_flat):
    nrow = idx2.shape[0]
    nch = nrow // _NTAB
    emb = 64
    emb_pad = emb + 1
    vocab = fused_flat.shape[0] // emb_pad
    seg = _NTAB * emb
    ch_per_w = nch // _NW
    groups = _CH // _LANES
    mesh = plsc.VectorSubcoreMesh(
        core_axis_name="c", subcore_axis_name="s",
        num_cores=_NUM_CORES, num_subcores=_NUM_SUBCORES,
    )
    out_type = jax.ShapeDtypeStruct((nch * seg, _CH), _F32)
    slab = ch_per_w
    scratch = [
        pltpu.VMEM((vocab * emb_pad,), _F32),
        pltpu.VMEM((slab * _NTAB, _CH), _I32),
        pltpu.VMEM((_NTAB * emb, _CH), _F32),
        pltpu.VMEM((_NTAB * emb, _CH), _F32),
        pltpu.SemaphoreType.DMA,
        pltpu.SemaphoreType.DMA,
    ]

    @functools.partial(pl.kernel, mesh=mesh, out_type=out_type,
                       scratch_types=scratch,
                       compiler_params=pltpu.CompilerParams(
                           use_tc_tiling_on_sc=False,
                           needs_layout_passes=False))
    def body(idx_h, ft_h, out_h, table_v, islab, ba, bb, sa, sb):
        wid = lax.axis_index("s") * _NUM_CORES + lax.axis_index("c")
        base = wid * ch_per_w
        pltpu.sync_copy(ft_h, table_v)

        def sub(j, jl, bufv, sem, guard):
            @pl.when(guard)
            def _():
                pltpu.make_async_copy(
                    bufv, out_h.at[pl.ds(0, seg)], sem).wait()

            def group(g, carry):
                for t in range(_NTAB):
                    rows = islab[jl * _NTAB + t, pl.ds(g * _LANES, _LANES)]
                    rs = rows * emb_pad
                    for c in range(emb):
                        v = plsc.load_gather(table_v, [rs + c])
                        bufv[t * emb + c, pl.ds(g * _LANES, _LANES)] = v
                return carry

            lax.fori_loop(0, groups, group, 0)
            pltpu.async_copy(bufv, out_h.at[pl.ds(j * seg, seg)], sem)

        def pair(k, carry):
            j0 = base + 2 * k
            jl0 = lax.rem(2 * k, slab)
            sub(j0, jl0, ba, sa, k >= 1)
            sub(j0 + 1, jl0 + 1, bb, sb, k >= 1)
            return carry

        def slab_loop(h, carry):
            pltpu.sync_copy(
                idx_h.at[pl.ds((base + h * slab) * _NTAB, slab * _NTAB)],
                islab)
            lax.fori_loop(h * slab // 2, (h + 1) * slab // 2, pair, 0)
            return carry

        lax.fori_loop(0, ch_per_w // slab, slab_loop, 0)
        pltpu.make_async_copy(ba, out_h.at[pl.ds(0, seg)], sa).wait()
        pltpu.make_async_copy(bb, out_h.at[pl.ds(0, seg)], sb).wait()

    return body(idx2, fused_flat)


def _epilogue_body(ws_ref, rt_ref, wr_ref, nw2_ref, wn_ref, nb_ref,
                   lb_ref, g_ref, bb_ref, xg_ref, rid_ref, n0_ref, n1_ref,
                   out_ref):
    seg = ws_ref.shape[0]
    kb = rid_ref.shape[1]
    tdot = lambda xt, w: lax.dot_general(
        xt, w, (((0,), (0,)), ((), ())), preferred_element_type=_F32)
    rw = jnp.dot(rt_ref[...], wr_ref[...], preferred_element_type=_F32)
    nw = jnp.dot(nw2_ref[...], wn_ref[...], preferred_element_type=_F32)
    brow = (jnp.dot(nb_ref[...], wn_ref[...], preferred_element_type=_F32)
            + lb_ref[...])
    for q in range(kb):
        x = xg_ref[q * seg:(q + 1) * seg, :]
        y = tdot(x, ws_ref[...])
        oht = (rid_ref[0, q:q + 1, :]
               == lax.broadcasted_iota(_I32, (4, _CH), 0)).astype(_F32)
        y = y + tdot(oht, rw)
        y = y + tdot(n0_ref[0, q:q + 1, :], nw[0:1, :])
        y = y + tdot(n1_ref[0, q:q + 1, :], nw[1:2, :])
        y = y + brow
        mu = jnp.mean(y, axis=1, keepdims=True)
        d = y - mu
        var = jnp.mean(d * d, axis=1, keepdims=True)
        out_ref[q * _CH:(q + 1) * _CH, :] = (
            d / jnp.sqrt(var + 1e-12) * g_ref[...] + bb_ref[...])


def kernel(input_ids, position_ids, timestamp, elapsed_time, response_table,
           num_W, num_b, bn_gamma, bn_beta, bn_mean, bn_var, elapsed_table,
           lag_table, pos_table, lin_W, lin_b, ln_gamma, ln_beta):
    b, l = input_ids.shape
    n = b * l
    hid = lin_W.shape[1]
    resp_w = response_table.shape[1]
    emb = lag_table.shape[1]
    nch = n // _CH
    seg = _NTAB * emb

    ts_f = timestamp.astype(_F32)
    bn = jnp.stack([bn_mean.astype(_F32), bn_var.astype(_F32),
                    bn_gamma.astype(_F32), bn_beta.astype(_F32)], axis=0)
    rb = 256
    bspec = pl.BlockSpec((rb, l), lambda i: (i, 0))
    i1, i2, i3, n0, n1 = pl.pallas_call(
        _prologue_body,
        grid=(b // rb,),
        in_specs=[pl.BlockSpec((4, 2), lambda i: (0, 0))] + [bspec] * 3,
        out_specs=[bspec] * 5,
        out_shape=[jax.ShapeDtypeStruct((b, l), _I32)] * 3
        + [jax.ShapeDtypeStruct((b, l), _F32)] * 2,
    )(bn, ts_f, elapsed_time.astype(_F32), position_ids.astype(_I32))

    idx2 = jnp.stack([x.reshape(nch, _CH) for x in (i1, i2, i3)],
                     axis=1).reshape(nch * _NTAB, _CH)
    fused_flat = jnp.pad(jnp.concatenate([
        lag_table.astype(_F32), elapsed_table.astype(_F32),
        pos_table.astype(_F32)], axis=0), ((0, 0), (0, 1))).reshape(-1)

    kb = 16
    ridt = input_ids.astype(_I32).reshape(nch, _CH)
    n0t = n0.reshape(nch, _CH)
    n1t = n1.reshape(nch, _CH)
    wr = lin_W[0:resp_w]
    wn = lin_W[resp_w:resp_w + emb]
    wstack = lin_W[resp_w + emb:resp_w + 4 * emb]

    const = lambda shape: pl.BlockSpec(shape, lambda i: (0, 0))
    weights = (wstack, response_table.astype(_F32), wr, num_W.astype(_F32),
               wn, num_b.reshape(1, emb), lin_b.reshape(1, hid),
               ln_gamma.reshape(1, hid), ln_beta.reshape(1, hid))

    def epilogue(gath_p, ridt_p, n0t_p, n1t_p):
        nch_p = ridt_p.shape[0]
        return pl.pallas_call(
            _epilogue_body,
            grid=(nch_p // kb,),
            in_specs=[const(w.shape) for w in weights]
            + [pl.BlockSpec((kb * seg, _CH), lambda i: (i, 0)),
               pl.BlockSpec((1, kb, _CH), lambda i: (i, 0, 0)),
               pl.BlockSpec((1, kb, _CH), lambda i: (i, 0, 0)),
               pl.BlockSpec((1, kb, _CH), lambda i: (i, 0, 0))],
            out_specs=pl.BlockSpec((kb * _CH, hid), lambda i: (i, 0)),
            out_shape=jax.ShapeDtypeStruct((nch_p * _CH, hid), _F32),
            compiler_params=pltpu.CompilerParams(
                fuse_transposed_lhs_in_matmul=True),
        )(*weights, gath_p,
          ridt_p.reshape(nch_p // kb, kb, _CH),
          n0t_p.reshape(nch_p // kb, kb, _CH),
          n1t_p.reshape(nch_p // kb, kb, _CH))

    cuts = (0, 18 * _NW, 34 * _NW, nch)
    outs = []
    parts = []
    for lo, hi in zip(cuts[:-1], cuts[1:]):
        parts.append((
            _sc_gather(idx2[lo * _NTAB:hi * _NTAB], fused_flat),
            ridt[lo:hi], n0t[lo:hi], n1t[lo:hi]))
    for p in parts:
        outs.append(epilogue(*p))
    out = jnp.concatenate(outs, axis=0)
    return out.reshape(b, l, hid)
